# Initial kernel scaffold; baseline (speedup 1.0000x reference)
import jax
import jax.numpy as jnp
from jax.experimental import pallas as pl


def kernel(x, edge_index, edge_weight, skip, H, C, params):
    raise NotImplementedError("write your pallas kernel here")


# speedup vs baseline: 13.8323x; 13.8323x over previous
import jax
import jax.numpy as jnp
from jax import lax
from jax.experimental import pallas as pl
from jax.experimental.pallas import tpu as pltpu
from jax.experimental.pallas import tpu_sc as plsc

F32 = jnp.float32

_NT = 16
_LANES = 16
_WS = 16
_CE = 1024
_SUB = 128
_RCH = 512


def _sc_spmm(table, src_flat, dst2, ew_flat, zrows, num_slabs, n_nodes, n_pad):
    e_pad = src_flat.shape[0]
    te = e_pad // _NT
    nch = te // _CE
    half = (num_slabs + 1) // 2
    nrch = n_pad // _RCH
    qmax = (nrch + _NT - 1) // _NT

    mesh = plsc.VectorSubcoreMesh(core_axis_name="c", subcore_axis_name="s")

    def body(table_hbm, src_hbm, dst_hbm, ew_hbm, z_hbm, out_hbm,
             src_v, idx_v, dst_v, ew_v, rows_v, zbuf, acc, sem):
        core = lax.axis_index("c")
        tile = lax.axis_index("s")
        pltpu.sync_copy(z_hbm, zbuf)

        for k in range(half):
            s = core * half + k

            @pl.when(s < num_slabs)
            def _slab():
                def zero_chunk(q, car):
                    c = tile + q * _NT

                    @pl.when(c < nrch)
                    def _():
                        pltpu.sync_copy(zbuf, acc.at[pl.ds(c * _RCH, _RCH)])
                    return car
                lax.fori_loop(0, qmax, zero_chunk, 0)
                plsc.subcore_barrier()

                off = (s * n_nodes).astype(jnp.int32) if hasattr(s, "astype") else jnp.int32(s * n_nodes)

                def edge_chunk(cidx, car):
                    base = tile * te + cidx * _CE
                    rbase = tile * (te // _SUB) + cidx * (_CE // _SUB)
                    pltpu.sync_copy(src_hbm.at[pl.ds(base, _CE)], src_v)
                    pltpu.sync_copy(ew_hbm.at[pl.ds(base, _CE)], ew_v)
                    pltpu.sync_copy(dst_hbm.at[pl.ds(rbase, _CE // _SUB)], dst_v)

                    def addidx(i, c2):
                        idx_v[pl.ds(i * _LANES, _LANES)] = (
                            src_v[pl.ds(i * _LANES, _LANES)] + off)
                        return c2
                    lax.fori_loop(0, _CE // _LANES, addidx, 0)

                    descs = [
                        pltpu.async_copy(
                            table_hbm.at[idx_v.at[pl.ds(j * _SUB, _SUB)]],
                            rows_v.at[pl.ds(j * _SUB, _SUB)], sem)
                        for j in range(_CE // _SUB)
                    ]
                    for d in descs:
                        d.wait()

                    def mul_group(g, c2):
                        ew16 = ew_v[pl.ds(g * _LANES, _LANES)]
                        for e in range(_LANES):
                            ea = g * _LANES + e
                            w = lax.gather(
                                ew16, jnp.full((_LANES, 1), e, jnp.int32),
                                dimension_numbers=lax.GatherDimensionNumbers(
                                    offset_dims=(), collapsed_slice_dims=(0,),
                                    start_index_map=(0,)),
                                slice_sizes=(1,),
                                mode=lax.GatherScatterMode.PROMISE_IN_BOUNDS)
                            rows_v[ea, pl.ds(0, _WS)] = (
                                rows_v[ea, pl.ds(0, _WS)] * w)
                        return c2
                    lax.fori_loop(0, _CE // _LANES, mul_group, 0)

                    for j in range(_CE // _SUB):
                        pltpu.sync_copy(rows_v.at[pl.ds(j * _SUB, _SUB)],
                                        acc.at[dst_v.at[j]], add=True)
                    return car
                lax.fori_loop(0, nch, edge_chunk, 0)
                plsc.subcore_barrier()

                def flush_chunk(q, car):
                    c = tile + q * _NT

                    @pl.when(c < nrch)
                    def _():
                        pltpu.sync_copy(acc.at[pl.ds(c * _RCH, _RCH)],
                                        out_hbm.at[s].at[pl.ds(c * _RCH, _RCH)])
                    return car
                lax.fori_loop(0, qmax, flush_chunk, 0)
                plsc.subcore_barrier()

    return pl.kernel(
        body,
        out_type=jax.ShapeDtypeStruct((num_slabs, n_pad, _WS), F32),
        mesh=mesh,
        compiler_params=pltpu.CompilerParams(use_tc_tiling_on_sc=False),
        scratch_types=[
            pltpu.VMEM((_CE,), jnp.int32),
            pltpu.VMEM((_CE,), jnp.int32),
            pltpu.VMEM((_CE // _SUB, _SUB), jnp.int32),
            pltpu.VMEM((_CE,), F32),
            pltpu.VMEM((_CE, _WS), F32),
            pltpu.VMEM((_RCH, _WS), F32),
            pltpu.VMEM_SHARED((n_pad, _WS), F32),
            pltpu.SemaphoreType.DMA,
        ],
    )(table, src_flat, dst2, ew_flat, zrows)


def _ln_b(v, g, b):
    mu = jnp.mean(v, axis=-1, keepdims=True)
    var = jnp.mean((v - mu) ** 2, axis=-1, keepdims=True)
    return (v - mu) * lax.rsqrt(var + 1e-5) * g + b


def _tc_layer0(s0, c0, ws, bs, gh, bh, gc, bcc, tn):
    n = s0.shape[0]
    kdim = s0.shape[1]
    grid = n // tn

    def body(s_ref, c_ref, wi, wf, wc, wo, bi, bf, bc_, bo,
             gh_r, bh_r, gc_r, bcc_r, hid_o, cel_o):
        s = s_ref[...]
        pi = jnp.dot(s, wi[...], preferred_element_type=F32) + bi[...]
        pf = jnp.dot(s, wf[...], preferred_element_type=F32) + bf[...]
        pc = jnp.dot(s, wc[...], preferred_element_type=F32) + bc_[...]
        po = jnp.dot(s, wo[...], preferred_element_type=F32) + bo[...]
        i_ = jax.nn.sigmoid(pi)
        f_ = jax.nn.sigmoid(pf)
        g_ = jnp.tanh(pc)
        o_ = jax.nn.sigmoid(po)
        cn = f_ * c_ref[...] + i_ * g_
        hn = o_ * jnp.tanh(cn)
        hid_o[...] = _ln_b(hn, gh_r[...], bh_r[...])
        cel_o[...] = _ln_b(cn, gc_r[...], bcc_r[...])

    row = lambda i: (i, 0)
    fix = lambda i: (0, 0)
    return pl.pallas_call(
        body,
        grid=(grid,),
        in_specs=[
            pl.BlockSpec((tn, kdim), row), pl.BlockSpec((tn, 64), row),
        ] + [pl.BlockSpec((kdim, 64), fix)] * 4
          + [pl.BlockSpec((1, 64), fix)] * 8,
        out_specs=[pl.BlockSpec((tn, 64), row)] * 2,
        out_shape=[jax.ShapeDtypeStruct((n, 64), F32)] * 2,
    )(s0, c0, *ws, *bs, gh, bh, gc, bcc)


def _tc_layer1(s1, c1, skip, ws, bs, gh, bh, gc, bcc, go, bo_ln,
               fa, fb, fbias, f2w, f2b, tn):
    n = s1.shape[0]
    kdim = s1.shape[1]
    grid = n // tn

    def body(s_ref, c_ref, sk_ref, wi, wf, wc, wo, bi, bf, bc_, bo,
             gh_r, bh_r, gc_r, bcc_r, go_r, bol_r,
             fa_r, fb_r, fbias_r, f2w_r, f2b_r,
             hid_o, cel_o, o_out):
        s = s_ref[...]
        pi = jnp.dot(s, wi[...], preferred_element_type=F32) + bi[...]
        pf = jnp.dot(s, wf[...], preferred_element_type=F32) + bf[...]
        pc = jnp.dot(s, wc[...], preferred_element_type=F32) + bc_[...]
        po = jnp.dot(s, wo[...], preferred_element_type=F32) + bo[...]
        i_ = jax.nn.sigmoid(pi)
        f_ = jax.nn.sigmoid(pf)
        g_ = jnp.tanh(pc)
        o_ = jax.nn.sigmoid(po)
        cn = f_ * c_ref[...] + i_ * g_
        hn = o_ * jnp.tanh(cn)
        hid_o[...] = _ln_b(hn, gh_r[...], bh_r[...])
        cel_o[...] = _ln_b(cn, gc_r[...], bcc_r[...])
        ob = jnp.maximum(_ln_b(hn, go_r[...], bol_r[...]), 0.0)
        t = (jnp.dot(ob, fa_r[...], preferred_element_type=F32)
             + sk_ref[...] * fb_r[...] + fbias_r[...])
        t = jnp.maximum(t, 0.0)
        o_out[...] = jax.nn.sigmoid(
            jnp.sum(t * f2w_r[...], axis=-1, keepdims=True) + f2b_r[...])

    row = lambda i: (i, 0)
    fix = lambda i: (0, 0)
    return pl.pallas_call(
        body,
        grid=(grid,),
        in_specs=[
            pl.BlockSpec((tn, kdim), row), pl.BlockSpec((tn, 64), row),
            pl.BlockSpec((tn, 1), row),
        ] + [pl.BlockSpec((kdim, 64), fix)] * 4
          + [pl.BlockSpec((1, 64), fix)] * 10
          + [pl.BlockSpec((64, 64), fix)]
          + [pl.BlockSpec((1, 64), fix)] * 3
          + [pl.BlockSpec((1, 1), fix)],
        out_specs=[pl.BlockSpec((tn, 64), row)] * 2 + [pl.BlockSpec((tn, 1), row)],
        out_shape=[jax.ShapeDtypeStruct((n, 64), F32)] * 2
                  + [jax.ShapeDtypeStruct((n, 1), F32)],
    )(s1, c1, skip, *ws, *bs, gh, bh, gc, bcc, go, bo_ln,
      fa, fb, fbias, f2w, f2b)


def kernel(x, edge_index, edge_weight, skip, H, C, params):
    n = x.shape[0]
    e = edge_index.shape[1]
    fin = x.shape[1]
    n_pad = ((n + _RCH - 1) // _RCH) * _RCH
    e_blk = _NT * _CE
    e_pad = ((e + e_blk - 1) // e_blk) * e_blk
    padw = e_pad - e

    src = edge_index[0]
    dst = edge_index[1]
    srcf = jnp.concatenate([src, jnp.zeros((padw,), jnp.int32)])
    dstf = jnp.concatenate([dst, jnp.zeros((padw,), jnp.int32)])
    ewf = jnp.concatenate([edge_weight, jnp.zeros((padw,), F32)])
    dst2 = dstf.reshape(-1, _SUB)
    zrows = jnp.zeros((_RCH, _WS), F32)

    def slabify(m):
        ns = m.shape[1] // _WS
        return m.reshape(n, ns, _WS).transpose(1, 0, 2).reshape(ns * n, _WS)

    H0, H1 = H[0], H[1]
    xpad = jnp.pad(x, ((0, 0), (0, _WS - fin)))
    t1 = jnp.concatenate([slabify(H0), slabify(H1), xpad], axis=0)
    S = _sc_spmm(t1, srcf, dst2, ewf, zrows, 9, n, n_pad)
    ah0 = jnp.concatenate([S[j, :n] for j in range(4)], axis=1)
    ah1 = jnp.concatenate([S[j, :n] for j in range(4, 8)], axis=1)
    ax = S[8, :n, :fin]
    s0cat = jnp.concatenate([ax, ah0], axis=1)

    l0, l1 = params['layers'][0], params['layers'][1]
    gates = ('i', 'f', 'c', 'o')
    ws0 = [jnp.concatenate([l0['Wx_' + g], l0['Wh_' + g]], axis=0) for g in gates]
    bs0 = [l0['b_' + g].reshape(1, 64) for g in gates]
    gh = params['ln_h_g'].reshape(1, 64)
    bh = params['ln_h_b'].reshape(1, 64)
    gc = params['ln_c_g'].reshape(1, 64)
    bcc = params['ln_c_b'].reshape(1, 64)

    tn = 2000
    hid0, cel0 = _tc_layer0(s0cat, C[0], ws0, bs0, gh, bh, gc, bcc, tn)

    t2 = slabify(hid0)
    S2 = _sc_spmm(t2, srcf, dst2, ewf, zrows, 4, n, n_pad)
    s1cat = jnp.concatenate([S2[j, :n] for j in range(4)] + [ah1], axis=1)

    ws1 = [jnp.concatenate([l1['Wx_' + g], l1['Wh_' + g]], axis=0) for g in gates]
    bs1 = [l1['b_' + g].reshape(1, 64) for g in gates]
    go = params['ln_o_g'].reshape(1, 64)
    bo_ln = params['ln_o_b'].reshape(1, 64)
    fa = params['fc1_W'][:64]
    fb = params['fc1_W'][64:65]
    fbias = params['fc1_b'].reshape(1, 64)
    f2w = params['fc2_W'].T
    f2b = params['fc2_b'].reshape(1, 1)

    hid1, cel1, o = _tc_layer1(s1cat, C[1], skip, ws1, bs1, gh, bh, gc, bcc,
                               go, bo_ln, fa, fb, fbias, f2w, f2b, tn)

    hidden = jnp.stack([hid0, hid1])
    cell = jnp.stack([cel0, cel1])
    return o, hidden, cell


# speedup vs baseline: 19.9189x; 1.4400x over previous
import jax
import jax.numpy as jnp
from jax import lax
from jax.experimental import pallas as pl
from jax.experimental.pallas import tpu as pltpu
from jax.experimental.pallas import tpu_sc as plsc

F32 = jnp.float32

_NT = 16
_LANES = 16
_WS = 16
_CE = 1024
_SUB = 128
_RCH = 512


def _splat(v16, e):
    return lax.gather(
        v16, jnp.full((_LANES, 1), e, jnp.int32),
        dimension_numbers=lax.GatherDimensionNumbers(
            offset_dims=(), collapsed_slice_dims=(0,), start_index_map=(0,)),
        slice_sizes=(1,),
        mode=lax.GatherScatterMode.PROMISE_IN_BOUNDS)


def _sc_spmm(table, src_flat, dst2, ew_flat, zrows, num_slabs, n_nodes, n_pad):
    e_pad = src_flat.shape[0]
    te = e_pad // _NT
    nch = te // _CE
    half = (num_slabs + 1) // 2
    nrch = n_pad // _RCH
    qmax = (nrch + _NT - 1) // _NT
    nsub = _CE // _SUB

    mesh = plsc.VectorSubcoreMesh(core_axis_name="c", subcore_axis_name="s")

    def body(table_hbm, src_hbm, dst_hbm, ew_hbm, z_hbm, out_hbm,
             src2, idx2, dst3, ew2, rows2, zbuf, acc, sem_st, sem_g, sem_sc):
        core = lax.axis_index("c")
        tile = lax.axis_index("s")
        pltpu.sync_copy(z_hbm, zbuf)

        def stage_issue(c):
            base = tile * te + c * _CE
            b = c % 2
            pltpu.async_copy(src_hbm.at[pl.ds(base, _CE)], src2.at[b], sem_st)
            pltpu.async_copy(ew_hbm.at[pl.ds(base, _CE)], ew2.at[b], sem_st)

        def stage_drain(c):
            base = tile * te + c * _CE
            b = c % 2
            pltpu.make_async_copy(
                src_hbm.at[pl.ds(base, _CE)], src2.at[b], sem_st).wait()
            pltpu.make_async_copy(
                ew_hbm.at[pl.ds(base, _CE)], ew2.at[b], sem_st).wait()

        def idx_compute(c, off):
            b = c % 2

            def f(i, car):
                idx2[b, pl.ds(i * _LANES, _LANES)] = (
                    src2[b, pl.ds(i * _LANES, _LANES)] + off)
                return car
            lax.fori_loop(0, _CE // _LANES, f, 0)

        def dst_stage(c):
            rbase = tile * (te // _SUB) + c * nsub
            pltpu.sync_copy(dst_hbm.at[pl.ds(rbase, nsub)], dst3.at[c % 2])

        def gathers_issue(c):
            b = c % 2
            for j in range(nsub):
                pltpu.async_copy(
                    table_hbm.at[idx2.at[b].at[pl.ds(j * _SUB, _SUB)]],
                    rows2.at[b].at[pl.ds(j * _SUB, _SUB)], sem_g)

        def gathers_drain(c):
            b = c % 2
            for j in range(nsub):
                pltpu.make_async_copy(
                    table_hbm.at[idx2.at[b].at[pl.ds(j * _SUB, _SUB)]],
                    rows2.at[b].at[pl.ds(j * _SUB, _SUB)], sem_g).wait()

        def scatters_issue(c):
            b = c % 2
            for j in range(nsub):
                pltpu.async_copy(rows2.at[b].at[pl.ds(j * _SUB, _SUB)],
                                 acc.at[dst3.at[b].at[j]], sem_sc, add=True)

        def scatters_drain(c):
            b = c % 2
            for j in range(nsub):
                pltpu.make_async_copy(rows2.at[b].at[pl.ds(j * _SUB, _SUB)],
                                      acc.at[dst3.at[b].at[j]], sem_sc).wait()

        def multiply(c):
            b = c % 2

            def mg(g, car):
                ew16 = ew2[b, pl.ds(g * _LANES, _LANES)]
                for e in range(_LANES):
                    ea = g * _LANES + e
                    w = _splat(ew16, e)
                    rows2[b, ea, pl.ds(0, _LANES)] = (
                        rows2[b, ea, pl.ds(0, _LANES)] * w)
                return car
            lax.fori_loop(0, _CE // _LANES, mg, 0)

        for k in range(half):
            s = core * half + k

            @pl.when(s < num_slabs)
            def _slab():
                def zero_chunk(q, car):
                    c = tile + q * _NT

                    @pl.when(c < nrch)
                    def _():
                        pltpu.sync_copy(zbuf, acc.at[pl.ds(c * _RCH, _RCH)])
                    return car
                lax.fori_loop(0, qmax, zero_chunk, 0)
                plsc.subcore_barrier()

                off = (s * n_nodes).astype(jnp.int32)

                stage_issue(0)
                stage_drain(0)
                idx_compute(0, off)
                dst_stage(0)
                gathers_issue(0)
                stage_issue(1)

                def step(i, car):
                    nxt = i + 1

                    @pl.when(nxt < nch)
                    def _pre():
                        stage_drain(nxt)
                        idx_compute(nxt, off)

                        @pl.when(i >= 1)
                        def _ds():
                            scatters_drain(nxt)
                        dst_stage(nxt)
                        gathers_issue(nxt)

                    gathers_drain(i)
                    multiply(i)
                    scatters_issue(i)

                    @pl.when(i + 2 < nch)
                    def _st():
                        stage_issue(i + 2)
                    return car
                lax.fori_loop(0, nch, step, 0)
                scatters_drain(nch - 2)
                scatters_drain(nch - 1)
                plsc.subcore_barrier()

                def flush_chunk(q, car):
                    c = tile + q * _NT

                    @pl.when(c < nrch)
                    def _():
                        pltpu.sync_copy(acc.at[pl.ds(c * _RCH, _RCH)],
                                        out_hbm.at[s].at[pl.ds(c * _RCH, _RCH)])
                    return car
                lax.fori_loop(0, qmax, flush_chunk, 0)
                plsc.subcore_barrier()

    return pl.kernel(
        body,
        out_type=jax.ShapeDtypeStruct((num_slabs, n_pad, _WS), F32),
        mesh=mesh,
        compiler_params=pltpu.CompilerParams(use_tc_tiling_on_sc=False),
        scratch_types=[
            pltpu.VMEM((2, _CE), jnp.int32),
            pltpu.VMEM((2, _CE), jnp.int32),
            pltpu.VMEM((2, _CE // _SUB, _SUB), jnp.int32),
            pltpu.VMEM((2, _CE), F32),
            pltpu.VMEM((2, _CE, _WS), F32),
            pltpu.VMEM((_RCH, _WS), F32),
            pltpu.VMEM_SHARED((n_pad, _WS), F32),
            pltpu.SemaphoreType.DMA,
            pltpu.SemaphoreType.DMA,
            pltpu.SemaphoreType.DMA,
        ],
    )(table, src_flat, dst2, ew_flat, zrows)


def _ln_b(v, g, b):
    mu = jnp.mean(v, axis=-1, keepdims=True)
    var = jnp.mean((v - mu) ** 2, axis=-1, keepdims=True)
    return (v - mu) * lax.rsqrt(var + 1e-5) * g + b


def _tc_layer0(s0, c0, ws, bs, gh, bh, gc, bcc, tn):
    n = s0.shape[0]
    kdim = s0.shape[1]
    grid = n // tn

    def body(s_ref, c_ref, wi, wf, wc, wo, bi, bf, bc_, bo,
             gh_r, bh_r, gc_r, bcc_r, hid_o, cel_o):
        s = s_ref[...]
        pi = jnp.dot(s, wi[...], preferred_element_type=F32) + bi[...]
        pf = jnp.dot(s, wf[...], preferred_element_type=F32) + bf[...]
        pc = jnp.dot(s, wc[...], preferred_element_type=F32) + bc_[...]
        po = jnp.dot(s, wo[...], preferred_element_type=F32) + bo[...]
        i_ = jax.nn.sigmoid(pi)
        f_ = jax.nn.sigmoid(pf)
        g_ = jnp.tanh(pc)
        o_ = jax.nn.sigmoid(po)
        cn = f_ * c_ref[...] + i_ * g_
        hn = o_ * jnp.tanh(cn)
        hid_o[...] = _ln_b(hn, gh_r[...], bh_r[...])
        cel_o[...] = _ln_b(cn, gc_r[...], bcc_r[...])

    row = lambda i: (i, 0)
    fix = lambda i: (0, 0)
    return pl.pallas_call(
        body,
        grid=(grid,),
        in_specs=[
            pl.BlockSpec((tn, kdim), row), pl.BlockSpec((tn, 64), row),
        ] + [pl.BlockSpec((kdim, 64), fix)] * 4
          + [pl.BlockSpec((1, 64), fix)] * 8,
        out_specs=[pl.BlockSpec((tn, 64), row)] * 2,
        out_shape=[jax.ShapeDtypeStruct((n, 64), F32)] * 2,
    )(s0, c0, *ws, *bs, gh, bh, gc, bcc)


def _tc_layer1(s1, c1, skip, ws, bs, gh, bh, gc, bcc, go, bo_ln,
               fa, fb, fbias, f2w, f2b, tn):
    n = s1.shape[0]
    kdim = s1.shape[1]
    grid = n // tn

    def body(s_ref, c_ref, sk_ref, wi, wf, wc, wo, bi, bf, bc_, bo,
             gh_r, bh_r, gc_r, bcc_r, go_r, bol_r,
             fa_r, fb_r, fbias_r, f2w_r, f2b_r,
             hid_o, cel_o, o_out):
        s = s_ref[...]
        pi = jnp.dot(s, wi[...], preferred_element_type=F32) + bi[...]
        pf = jnp.dot(s, wf[...], preferred_element_type=F32) + bf[...]
        pc = jnp.dot(s, wc[...], preferred_element_type=F32) + bc_[...]
        po = jnp.dot(s, wo[...], preferred_element_type=F32) + bo[...]
        i_ = jax.nn.sigmoid(pi)
        f_ = jax.nn.sigmoid(pf)
        g_ = jnp.tanh(pc)
        o_ = jax.nn.sigmoid(po)
        cn = f_ * c_ref[...] + i_ * g_
        hn = o_ * jnp.tanh(cn)
        hid_o[...] = _ln_b(hn, gh_r[...], bh_r[...])
        cel_o[...] = _ln_b(cn, gc_r[...], bcc_r[...])
        ob = jnp.maximum(_ln_b(hn, go_r[...], bol_r[...]), 0.0)
        t = (jnp.dot(ob, fa_r[...], preferred_element_type=F32)
             + sk_ref[...] * fb_r[...] + fbias_r[...])
        t = jnp.maximum(t, 0.0)
        o_out[...] = jax.nn.sigmoid(
            jnp.sum(t * f2w_r[...], axis=-1, keepdims=True) + f2b_r[...])

    row = lambda i: (i, 0)
    fix = lambda i: (0, 0)
    return pl.pallas_call(
        body,
        grid=(grid,),
        in_specs=[
            pl.BlockSpec((tn, kdim), row), pl.BlockSpec((tn, 64), row),
            pl.BlockSpec((tn, 1), row),
        ] + [pl.BlockSpec((kdim, 64), fix)] * 4
          + [pl.BlockSpec((1, 64), fix)] * 10
          + [pl.BlockSpec((64, 64), fix)]
          + [pl.BlockSpec((1, 64), fix)] * 3
          + [pl.BlockSpec((1, 1), fix)],
        out_specs=[pl.BlockSpec((tn, 64), row)] * 2 + [pl.BlockSpec((tn, 1), row)],
        out_shape=[jax.ShapeDtypeStruct((n, 64), F32)] * 2
                  + [jax.ShapeDtypeStruct((n, 1), F32)],
    )(s1, c1, skip, *ws, *bs, gh, bh, gc, bcc, go, bo_ln,
      fa, fb, fbias, f2w, f2b)


def kernel(x, edge_index, edge_weight, skip, H, C, params):
    n = x.shape[0]
    e = edge_index.shape[1]
    fin = x.shape[1]
    n_pad = ((n + _RCH - 1) // _RCH) * _RCH
    e_blk = _NT * _CE
    e_pad = ((e + e_blk - 1) // e_blk) * e_blk
    padw = e_pad - e

    src = edge_index[0]
    dst = edge_index[1]
    srcf = jnp.concatenate([src, jnp.zeros((padw,), jnp.int32)])
    dstf = jnp.concatenate([dst, jnp.zeros((padw,), jnp.int32)])
    ewf = jnp.concatenate([edge_weight, jnp.zeros((padw,), F32)])
    dst2 = dstf.reshape(-1, _SUB)
    zrows = jnp.zeros((_RCH, _WS), F32)

    def slabify(m):
        ns = m.shape[1] // _WS
        return m.reshape(n, ns, _WS).transpose(1, 0, 2).reshape(ns * n, _WS)

    H0, H1 = H[0], H[1]
    xpad = jnp.pad(x, ((0, 0), (0, _WS - fin)))
    t1 = jnp.concatenate([slabify(H0), slabify(H1), xpad], axis=0)
    S = _sc_spmm(t1, srcf, dst2, ewf, zrows, 9, n, n_pad)
    ah0 = jnp.concatenate([S[j, :n] for j in range(4)], axis=1)
    ah1 = jnp.concatenate([S[j, :n] for j in range(4, 8)], axis=1)
    ax = S[8, :n, :fin]
    s0cat = jnp.concatenate([ax, ah0], axis=1)

    l0, l1 = params['layers'][0], params['layers'][1]
    gates = ('i', 'f', 'c', 'o')
    ws0 = [jnp.concatenate([l0['Wx_' + g], l0['Wh_' + g]], axis=0) for g in gates]
    bs0 = [l0['b_' + g].reshape(1, 64) for g in gates]
    gh = params['ln_h_g'].reshape(1, 64)
    bh = params['ln_h_b'].reshape(1, 64)
    gc = params['ln_c_g'].reshape(1, 64)
    bcc = params['ln_c_b'].reshape(1, 64)

    tn = 2000
    hid0, cel0 = _tc_layer0(s0cat, C[0], ws0, bs0, gh, bh, gc, bcc, tn)

    t2 = slabify(hid0)
    S2 = _sc_spmm(t2, srcf, dst2, ewf, zrows, 4, n, n_pad)
    s1cat = jnp.concatenate([S2[j, :n] for j in range(4)] + [ah1], axis=1)

    ws1 = [jnp.concatenate([l1['Wx_' + g], l1['Wh_' + g]], axis=0) for g in gates]
    bs1 = [l1['b_' + g].reshape(1, 64) for g in gates]
    go = params['ln_o_g'].reshape(1, 64)
    bo_ln = params['ln_o_b'].reshape(1, 64)
    fa = params['fc1_W'][:64]
    fb = params['fc1_W'][64:65]
    fbias = params['fc1_b'].reshape(1, 64)
    f2w = params['fc2_W'].T
    f2b = params['fc2_b'].reshape(1, 1)

    hid1, cel1, o = _tc_layer1(s1cat, C[1], skip, ws1, bs1, gh, bh, gc, bcc,
                               go, bo_ln, fa, fb, fbias, f2w, f2b, tn)

    hidden = jnp.stack([hid0, hid1])
    cell = jnp.stack([cel0, cel1])
    return o, hidden, cell
